# baseline (speedup 1.0000x reference)
import jax
import jax.numpy as jnp
from jax import lax
from jax.experimental import pallas as pl
from jax.experimental.pallas import tpu as pltpu
from jax.experimental.pallas import tpu_sc as plsc

B, S, E = 4, 8192, 128
NW = 32
SW = S // NW
CHUNK = 128
HB = SW // CHUNK
NCH = B * HB
NBUF = 4
LANES = 16


def _emb_body(tok_hbm, tok_emb_hbm, pos_emb_hbm, out_hbm,
              idx_v, pos_v, bufs, psem, gsems, ssems):
    c = lax.axis_index("c")
    s = lax.axis_index("s")
    wid = s * 2 + c
    s0 = wid * SW

    pos_d = pltpu.async_copy(pos_emb_hbm.at[pl.ds(s0, SW)], pos_v, psem)

    pltpu.sync_copy(tok_hbm.at[pl.ds(0, B), pl.ds(s0, SW)], idx_v)

    def start_gather(j):
        b, h = divmod(j, HB)
        return pltpu.async_copy(
            tok_emb_hbm.at[idx_v.at[b, pl.ds(h * CHUNK, CHUNK)]],
            bufs[j % NBUF], gsems[j % NBUF])

    def start_store(j):
        b, h = divmod(j, HB)
        return pltpu.async_copy(
            bufs[j % NBUF], out_hbm.at[b, pl.ds(s0 + h * CHUNK, CHUNK)],
            ssems[j % NBUF])

    gat_d = [None] * NCH
    st_d = [None] * NCH

    for j in range(NBUF):
        gat_d[j] = start_gather(j)

    pos_d.wait()
    for j in range(NCH):
        gat_d[j].wait()
        if j >= 2 and (j - 2) + NBUF < NCH:
            st_d[j - 2].wait()
            gat_d[(j - 2) + NBUF] = start_gather((j - 2) + NBUF)
        buf = bufs[j % NBUF]
        h0 = (j % HB) * CHUNK

        @plsc.parallel_loop(0, CHUNK, step=1, unroll=1)
        def add_pos(r):
            for k in range(E // LANES):
                sl = pl.ds(k * LANES, LANES)
                buf[r, sl] = buf[r, sl] + pos_v[h0 + r, sl]

        st_d[j] = start_store(j)

    for j in range(NCH):
        if j + NBUF >= NCH:
            st_d[j].wait()


@jax.jit
def _emb(tokens, tok_emb, pos_emb):
    mesh = plsc.VectorSubcoreMesh(core_axis_name="c", subcore_axis_name="s")
    run = pl.kernel(
        _emb_body,
        out_type=jax.ShapeDtypeStruct((B, S, E), jnp.float32),
        mesh=mesh,
        scratch_types=[
            pltpu.VMEM((B, SW), jnp.int32),
            pltpu.VMEM((SW, E), jnp.float32),
            [pltpu.VMEM((CHUNK, E), jnp.float32) for _ in range(NBUF)],
            pltpu.SemaphoreType.DMA,
            [pltpu.SemaphoreType.DMA for _ in range(NBUF)],
            [pltpu.SemaphoreType.DMA for _ in range(NBUF)],
        ],
    )
    return run(tokens, tok_emb, pos_emb)


def kernel(tokens, tok_emb, pos_emb):
    return _emb(tokens.astype(jnp.int32), tok_emb, pos_emb)


# speedup vs baseline: 1.0260x; 1.0260x over previous
import jax
import jax.numpy as jnp
from jax import lax
from jax.experimental import pallas as pl
from jax.experimental.pallas import tpu as pltpu
from jax.experimental.pallas import tpu_sc as plsc

B, S, E = 4, 8192, 128
NW = 32
SW = S // NW
CHUNK = 128
HB = SW // CHUNK
NCH = B * HB
NBUF = 5
LANES = 16


def _emb_body(tok_hbm, tok_emb_hbm, pos_emb_hbm, out_hbm,
              idx_v, pos_v, bufs, psem, gsems, ssems):
    c = lax.axis_index("c")
    s = lax.axis_index("s")
    wid = s * 2 + c
    s0 = wid * SW

    pos_d = pltpu.async_copy(pos_emb_hbm.at[pl.ds(s0, SW)], pos_v, psem)

    pltpu.sync_copy(tok_hbm.at[pl.ds(0, B), pl.ds(s0, SW)], idx_v)

    def start_gather(j):
        b, h = divmod(j, HB)
        return pltpu.async_copy(
            tok_emb_hbm.at[idx_v.at[b, pl.ds(h * CHUNK, CHUNK)]],
            bufs[j % NBUF], gsems[j % NBUF])

    def start_store(j):
        b, h = divmod(j, HB)
        return pltpu.async_copy(
            bufs[j % NBUF], out_hbm.at[b, pl.ds(s0 + h * CHUNK, CHUNK)],
            ssems[j % NBUF])

    gat_d = [None] * NCH
    st_d = [None] * NCH

    for j in range(NBUF):
        gat_d[j] = start_gather(j)

    pos_d.wait()
    for j in range(NCH):
        gat_d[j].wait()
        if j >= 2 and (j - 2) + NBUF < NCH:
            st_d[j - 2].wait()
            gat_d[(j - 2) + NBUF] = start_gather((j - 2) + NBUF)
        buf = bufs[j % NBUF]
        h0 = (j % HB) * CHUNK

        @plsc.parallel_loop(0, CHUNK, step=1, unroll=1)
        def add_pos(r):
            for k in range(E // LANES):
                sl = pl.ds(k * LANES, LANES)
                plsc.addupdate(buf.at[r, sl], pos_v[h0 + r, sl])

        st_d[j] = start_store(j)

    for j in range(NCH):
        if j + NBUF >= NCH:
            st_d[j].wait()


@jax.jit
def _emb(tokens, tok_emb, pos_emb):
    mesh = plsc.VectorSubcoreMesh(core_axis_name="c", subcore_axis_name="s")
    run = pl.kernel(
        _emb_body,
        out_type=jax.ShapeDtypeStruct((B, S, E), jnp.float32),
        mesh=mesh,
        scratch_types=[
            pltpu.VMEM((B, SW), jnp.int32),
            pltpu.VMEM((SW, E), jnp.float32),
            [pltpu.VMEM((CHUNK, E), jnp.float32) for _ in range(NBUF)],
            pltpu.SemaphoreType.DMA,
            [pltpu.SemaphoreType.DMA for _ in range(NBUF)],
            [pltpu.SemaphoreType.DMA for _ in range(NBUF)],
        ],
    )
    return run(tokens, tok_emb, pos_emb)


def kernel(tokens, tok_emb, pos_emb):
    return _emb(tokens.astype(jnp.int32), tok_emb, pos_emb)
